# baseline (speedup 1.0000x reference)
import functools

import jax
import jax.numpy as jnp
from jax.experimental import pallas as pl
from jax.experimental.pallas import tpu as pltpu

_EPS = 1e-5
_TILE_N = 512


def _bn_stream_kernel(x_ref, g_ref, b_ref, y_ref, xs_ref, s_ref, ss_ref, *,
                      num_n, tile_n, inv_n, eps):
    i = pl.program_id(1)

    @pl.when(i == 0)
    def _():
        s_ref[...] = jnp.zeros_like(s_ref)
        ss_ref[...] = jnp.zeros_like(ss_ref)

    x = x_ref[...].astype(jnp.float32)
    s_ref[...] += jnp.sum(x, axis=0, keepdims=True)
    ss_ref[...] += jnp.sum(x * x, axis=0, keepdims=True)
    xs_ref[pl.ds(i * tile_n, tile_n), :] = x

    @pl.when(i == num_n - 1)
    def _():
        inv = jnp.float32(inv_n)
        m1 = s_ref[...] * inv
        var = jnp.maximum(ss_ref[...] * inv - m1 * m1, 0.0)
        k = g_ref[...] * jax.lax.rsqrt(var + eps)
        y_ref[...] = ((xs_ref[...] - m1) * k + b_ref[...]).astype(y_ref.dtype)


def kernel(x, gamma, beta):
    n, c = x.shape
    g2d = gamma.astype(jnp.float32).reshape(1, c)
    b2d = beta.astype(jnp.float32).reshape(1, c)

    tile_c = 128 if c % 128 == 0 else c
    num_strips = c // tile_c
    tile_n = _TILE_N if (n % _TILE_N == 0 and n > _TILE_N) else n
    num_n = n // tile_n

    body = functools.partial(
        _bn_stream_kernel,
        num_n=num_n, tile_n=tile_n, inv_n=1.0 / n, eps=_EPS)
    return pl.pallas_call(
        body,
        out_shape=jax.ShapeDtypeStruct((n, c), x.dtype),
        grid=(num_strips, num_n),
        in_specs=[
            pl.BlockSpec((tile_n, tile_c), lambda j, i: (i, j)),
            pl.BlockSpec((1, tile_c), lambda j, i: (0, j)),
            pl.BlockSpec((1, tile_c), lambda j, i: (0, j)),
        ],
        out_specs=pl.BlockSpec((n, tile_c), lambda j, i: (0, j)),
        scratch_shapes=[
            pltpu.VMEM((n, tile_c), jnp.float32),
            pltpu.VMEM((1, tile_c), jnp.float32),
            pltpu.VMEM((1, tile_c), jnp.float32),
        ],
        compiler_params=pltpu.CompilerParams(
            dimension_semantics=("parallel", "arbitrary"),
            vmem_limit_bytes=56 * 1024 * 1024,
        ),
    )(x, g2d, b2d)


# speedup vs baseline: 3.2505x; 3.2505x over previous
import functools

import jax
import jax.numpy as jnp
from jax.experimental import pallas as pl
from jax.experimental.pallas import tpu as pltpu

_EPS = 1e-5


def _bn_strip_kernel(x_ref, g_ref, b_ref, y_ref, *, inv_n, eps):
    x = x_ref[...].astype(jnp.float32)
    inv = jnp.float32(inv_n)
    m1 = jnp.sum(x, axis=0, keepdims=True) * inv
    m2 = jnp.sum(x * x, axis=0, keepdims=True) * inv
    var = jnp.maximum(m2 - m1 * m1, 0.0)
    k = g_ref[...] * jax.lax.rsqrt(var + eps)
    y_ref[...] = ((x - m1) * k + b_ref[...]).astype(y_ref.dtype)


def _strip_width(n, c, itemsize):
    if c % 128 != 0:
        return c
    w = 128
    while w < c and n * 2 * w * (2 * itemsize + 2 * itemsize + 8) > 56 * 1024 * 1024:
        break
    return w


def kernel(x, gamma, beta):
    n, c = x.shape
    g2d = gamma.astype(jnp.float32).reshape(1, c)
    b2d = beta.astype(jnp.float32).reshape(1, c)

    tile_c = _strip_width(n, c, x.dtype.itemsize)
    num_strips = c // tile_c

    body = functools.partial(_bn_strip_kernel, inv_n=1.0 / n, eps=_EPS)
    return pl.pallas_call(
        body,
        out_shape=jax.ShapeDtypeStruct((n, c), x.dtype),
        grid=(num_strips,),
        in_specs=[
            pl.BlockSpec((n, tile_c), lambda j: (0, j)),
            pl.BlockSpec((1, tile_c), lambda j: (0, j)),
            pl.BlockSpec((1, tile_c), lambda j: (0, j)),
        ],
        out_specs=pl.BlockSpec((n, tile_c), lambda j: (0, j)),
        compiler_params=pltpu.CompilerParams(
            dimension_semantics=("parallel",),
            vmem_limit_bytes=56 * 1024 * 1024,
        ),
    )(x, g2d, b2d)


# speedup vs baseline: 3.5358x; 1.0878x over previous
import functools

import jax
import jax.numpy as jnp
from jax.experimental import pallas as pl
from jax.experimental.pallas import tpu as pltpu

_EPS = 1e-5


def _bn_strip_kernel(x_ref, g_ref, b_ref, y_ref, *, inv_n, eps):
    x = x_ref[...].astype(jnp.float32)
    inv = jnp.float32(inv_n)
    m1 = jnp.sum(x, axis=0, keepdims=True) * inv
    m2 = jnp.sum(x * x, axis=0, keepdims=True) * inv
    var = jnp.maximum(m2 - m1 * m1, 0.0)
    k = g_ref[...] * jax.lax.rsqrt(var + eps)
    y_ref[...] = ((x - m1) * k + b_ref[...]).astype(y_ref.dtype)


def _strip_width(n, c, itemsize):
    if c % 128 != 0:
        return c
    w = 256 if c % 256 == 0 else 128
    while w < c and n * 2 * w * (2 * itemsize + 2 * itemsize + 8) > 56 * 1024 * 1024:
        break
    return w


def kernel(x, gamma, beta):
    n, c = x.shape
    g2d = gamma.astype(jnp.float32).reshape(1, c)
    b2d = beta.astype(jnp.float32).reshape(1, c)

    tile_c = _strip_width(n, c, x.dtype.itemsize)
    num_strips = c // tile_c

    body = functools.partial(_bn_strip_kernel, inv_n=1.0 / n, eps=_EPS)
    return pl.pallas_call(
        body,
        out_shape=jax.ShapeDtypeStruct((n, c), x.dtype),
        grid=(num_strips,),
        in_specs=[
            pl.BlockSpec((n, tile_c), lambda j: (0, j)),
            pl.BlockSpec((1, tile_c), lambda j: (0, j)),
            pl.BlockSpec((1, tile_c), lambda j: (0, j)),
        ],
        out_specs=pl.BlockSpec((n, tile_c), lambda j: (0, j)),
        compiler_params=pltpu.CompilerParams(
            dimension_semantics=("parallel",),
            vmem_limit_bytes=56 * 1024 * 1024,
        ),
    )(x, g2d, b2d)
